# baseline (speedup 1.0000x reference)
import jax
import jax.numpy as jnp
from jax.experimental import pallas as pl
from jax.experimental.pallas import tpu as pltpu

_BM = 4096


_DN = (((1,), (1,)), ((), ()))


def _gate_kernel(x_ref, w_ref, out_ref):
    x = x_ref[...]
    w = w_ref[...]
    xh = x.astype(jnp.bfloat16)
    xl = (x - xh.astype(jnp.float32)).astype(jnp.bfloat16)
    wh = w.astype(jnp.bfloat16)
    wl = (w - wh.astype(jnp.float32)).astype(jnp.bfloat16)
    acc = jax.lax.dot_general(xh, wh, _DN, preferred_element_type=jnp.float32)
    acc += jax.lax.dot_general(xl, wh, _DN, preferred_element_type=jnp.float32)
    acc += jax.lax.dot_general(xh, wl, _DN, preferred_element_type=jnp.float32)
    out_ref[...] = acc


def kernel(x, W):
    m, d = x.shape
    e = W.shape[0]
    grid = (m // _BM,)
    logits = pl.pallas_call(
        _gate_kernel,
        grid=grid,
        in_specs=[
            pl.BlockSpec((_BM, d), lambda i: (i, 0)),
            pl.BlockSpec((e, d), lambda i: (0, 0)),
        ],
        out_specs=pl.BlockSpec((_BM, e), lambda i: (i, 0)),
        out_shape=jax.ShapeDtypeStruct((m, e), jnp.float32),
        compiler_params=pltpu.CompilerParams(
            dimension_semantics=("parallel",),
        ),
    )(x, W)
    return (logits, 2)
